# Initial kernel scaffold; baseline (speedup 1.0000x reference)
import jax
import jax.numpy as jnp
from jax.experimental import pallas as pl


def kernel(x, pos, batch, W1a, b1a, W1b, b1b, W2a, b2a, W2b, b2b, W3a, b3a, W3b, b3b, Wr, br):
    raise NotImplementedError("write your pallas kernel here")


# speedup vs baseline: 30.1376x; 30.1376x over previous
import functools
import math

import jax
import jax.numpy as jnp
from jax import lax
from jax.experimental import pallas as pl
from jax.experimental.pallas import tpu as pltpu
from jax.experimental.pallas import tpu_sc as plsc

B = 8
N = 10000
NEG = -1e30


def _knn_body(n_pad, n_valid, k, pcol_ref, prow_ref, out_ref):
    b = pl.program_id(0)
    pc = pcol_ref[0]
    pr = prow_ref[0]
    dx = pc[:, 0:1] - pr[0:1, :]
    dy = pc[:, 1:2] - pr[1:2, :]
    dz = pc[:, 2:3] - pr[2:3, :]
    d2 = dx * dx + dy * dy + dz * dz
    lane = lax.broadcasted_iota(jnp.int32, (n_pad, n_pad), 1)
    d2 = jnp.where(lane >= n_valid, jnp.inf, d2)
    off = b * n_pad
    for kk in range(k):
        minv = jnp.min(d2, axis=1, keepdims=True)
        idx = jnp.min(jnp.where(d2 == minv, lane, n_pad), axis=1, keepdims=True)
        out_ref[0, :, kk:kk + 1] = idx + off
        d2 = jnp.where(lane == idx, jnp.inf, d2)
    for kk in range(k, 8):
        out_ref[0, :, kk:kk + 1] = jnp.full((n_pad, 1), off, jnp.int32)


def _prep_body(hp_ref, wd_ref, ws_ref, ba_ref, a_ref, b_ref):
    hp = hp_ref[...]
    a_ref[...] = (jnp.dot(hp, wd_ref[...], preferred_element_type=jnp.float32)
                  + ba_ref[...])
    b_ref[...] = jnp.dot(hp, ws_ref[...], preferred_element_type=jnp.float32)


def _mlp_body(k, a_ref, be_ref, w2_ref, b2_ref, out_ref):
    a = a_ref[...]
    w2 = w2_ref[...]
    b2 = b2_ref[...]
    acc = None
    for j in range(k):
        t = jnp.maximum(a + be_ref[:, j * 32:(j + 1) * 32], 0.0)
        m = jnp.dot(t, w2, preferred_element_type=jnp.float32) + b2
        acc = m if acc is None else jnp.maximum(acc, m)
    out_ref[...] = jnp.maximum(acc, 0.0)


def _fps_body(n_pad, n_valid, m, m_pad, pxyz_ref, out_ref):
    px = pxyz_ref[0]
    py = pxyz_ref[1]
    pz = pxyz_ref[2]
    lane = lax.broadcasted_iota(jnp.int32, (B, n_pad), 1)
    offs = lax.broadcasted_iota(jnp.int32, (B, 1), 0) * n_pad
    d0x = px - px[:, 0:1]
    d0y = py - py[:, 0:1]
    d0z = pz - pz[:, 0:1]
    dmin0 = d0x * d0x + d0y * d0y + d0z * d0z
    dmin0 = jnp.where(lane >= n_valid, -1.0, dmin0)
    mlane = lax.broadcasted_iota(jnp.int32, (B, m_pad), 1)
    idxs0 = jnp.zeros((B, m_pad), jnp.int32)

    def body(i, carry):
        dmin, idxs = carry
        maxv = jnp.max(dmin, axis=1, keepdims=True)
        idxm = jnp.min(jnp.where(dmin == maxv, lane, n_pad), axis=1,
                       keepdims=True)
        idxs = jnp.where(mlane == i, idxm, idxs)
        sel = lane == idxm
        sx = jnp.sum(jnp.where(sel, px, 0.0), axis=1, keepdims=True)
        sy = jnp.sum(jnp.where(sel, py, 0.0), axis=1, keepdims=True)
        sz = jnp.sum(jnp.where(sel, pz, 0.0), axis=1, keepdims=True)
        ddx = px - sx
        ddy = py - sy
        ddz = pz - sz
        d = ddx * ddx + ddy * ddy + ddz * ddz
        return jnp.minimum(dmin, d), idxs

    _, idxs = lax.fori_loop(1, m, body, (dmin0, idxs0))
    out_ref[...] = idxs + offs


def _final_body(m_pad, n_valid, h_ref, wr_ref, br_ref, out_ref):
    wr = wr_ref[...]
    br = br_ref[...]
    sub = lax.broadcasted_iota(jnp.int32, (m_pad, 32), 0)
    rows = []
    for b in range(B):
        h = h_ref[b * m_pad:(b + 1) * m_pad, :]
        h = jnp.where(sub >= n_valid, NEG, h)
        rows.append(jnp.max(h, axis=0, keepdims=True))
    hp = jnp.concatenate(rows, axis=0)
    out_ref[...] = jnp.dot(hp, wr, preferred_element_type=jnp.float32) + br


def _knn(pcol, prow, n_pad, n_valid, k):
    return pl.pallas_call(
        functools.partial(_knn_body, n_pad, n_valid, k),
        grid=(B,),
        in_specs=[
            pl.BlockSpec((1, n_pad, 4), lambda b: (b, 0, 0)),
            pl.BlockSpec((1, 8, n_pad), lambda b: (b, 0, 0)),
        ],
        out_specs=pl.BlockSpec((1, n_pad, 8), lambda b: (b, 0, 0)),
        out_shape=jax.ShapeDtypeStruct((B, n_pad, 8), jnp.int32),
    )(pcol, prow)


def _prep(hp, wd, ws, ba):
    r, f = hp.shape
    return pl.pallas_call(
        _prep_body,
        out_shape=(jax.ShapeDtypeStruct((r, 32), jnp.float32),
                   jax.ShapeDtypeStruct((r, 32), jnp.float32)),
    )(hp, wd, ws, ba)


def _mlp(a, be, w2, b2, k):
    r = a.shape[0]
    return pl.pallas_call(
        functools.partial(_mlp_body, k),
        out_shape=jax.ShapeDtypeStruct((r, 32), jnp.float32),
    )(a, be, w2, b2)


def _fps(pxyz, n_pad, n_valid, m, m_pad):
    return pl.pallas_call(
        functools.partial(_fps_body, n_pad, n_valid, m, m_pad),
        out_shape=jax.ShapeDtypeStruct((B, m_pad), jnp.int32),
    )(pxyz)


def _final(h, wr, br, m_pad, n_valid):
    return pl.pallas_call(
        functools.partial(_final_body, m_pad, n_valid),
        out_shape=jax.ShapeDtypeStruct((B, 10), jnp.float32),
    )(h, wr, br)


@functools.lru_cache(maxsize=None)
def _sc_gather_call(v, d, e):
    info = plsc.get_sparse_core_info()
    nw = info.num_cores * info.num_subcores
    b_per_w = e // nw
    mesh = plsc.VectorSubcoreMesh(core_axis_name="c", subcore_axis_name="s")

    @functools.partial(
        pl.kernel, mesh=mesh,
        out_type=jax.ShapeDtypeStruct((e, d), jnp.float32),
        compiler_params=pltpu.CompilerParams(use_tc_tiling_on_sc=False),
        scratch_types=[
            pltpu.VMEM((b_per_w,), jnp.int32),
            pltpu.VMEM((b_per_w, d), jnp.float32),
            pltpu.SemaphoreType.DMA,
        ],
    )
    def gather(table_hbm, idx_hbm, out_hbm, idx_v, rows_v, sem):
        wid = lax.axis_index("s") * info.num_cores + lax.axis_index("c")
        base = wid * b_per_w
        pltpu.sync_copy(idx_hbm.at[pl.ds(base, b_per_w)], idx_v)
        pltpu.async_copy(table_hbm.at[idx_v], rows_v, sem).wait()
        pltpu.sync_copy(rows_v, out_hbm.at[pl.ds(base, b_per_w)])

    return gather


def _sc_gather(table, idx):
    v, d = table.shape
    return _sc_gather_call(v, d, idx.shape[0])(table, idx)


def _level(hp, pcol, prow, n_pad, n_valid, k, w1, b1, w2, b2):
    f = hp.shape[1] - 3
    wd = jnp.concatenate([w1[:f], -w1[f + f:]], axis=0)
    ws = w1[f:]
    src = _knn(pcol, prow, n_pad, n_valid, k)
    a, bfeat = _prep(hp, wd, ws, b1.reshape(1, 32))
    idx = src[:, :, :k].reshape(-1)
    be = _sc_gather(bfeat, idx)
    be = be.reshape(B * n_pad, k * 32)
    return _mlp(a, be, w2, b2.reshape(1, 32), k)


def _pack_level(pos_flat, n_pad):
    p = pos_flat.reshape(B, n_pad, 3)
    pcol = jnp.pad(p, ((0, 0), (0, 0), (0, 1)))
    prow = jnp.pad(p.transpose(0, 2, 1), ((0, 0), (0, 5), (0, 0)))
    pxyz = p.transpose(2, 0, 1)
    return pcol, prow, pxyz


def kernel(x, pos, batch, W1a, b1a, W1b, b1b, W2a, b2a, W2b, b2b,
           W3a, b3a, W3b, b3b, Wr, br):
    n1, n1p, k1 = N // B, 1280, 6
    m1 = math.ceil(n1 * 0.5)
    m1p = 640
    m2 = math.ceil(m1 * 0.5)
    m2p = 320
    k2, k3 = 4, 3

    posb = jnp.pad(pos.reshape(B, n1, 3), ((0, 0), (0, n1p - n1), (0, 0)))
    xb = jnp.pad(x.reshape(B, n1, 1), ((0, 0), (0, n1p - n1), (0, 0)))
    pos1 = posb.reshape(B * n1p, 3)
    hp1 = jnp.concatenate([xb.reshape(B * n1p, 1), pos1], axis=1)

    pcol1, prow1, pxyz1 = _pack_level(pos1, n1p)
    h1 = _level(hp1, pcol1, prow1, n1p, n1, k1, W1a, b1a, W1b, b1b)

    idx1 = _fps(pxyz1, n1p, n1, m1, m1p).reshape(-1)
    tab1 = jnp.concatenate(
        [pos1, h1, jnp.zeros((B * n1p, 13), jnp.float32)], axis=1)
    g1 = _sc_gather(tab1, idx1)
    pos2 = g1[:, :3]
    hp2 = jnp.concatenate([g1[:, 3:35], pos2], axis=1)

    pcol2, prow2, pxyz2 = _pack_level(pos2, m1p)
    h2 = _level(hp2, pcol2, prow2, m1p, m1, k2, W2a, b2a, W2b, b2b)

    idx2 = _fps(pxyz2, m1p, m1, m2, m2p).reshape(-1)
    tab2 = jnp.concatenate(
        [pos2, h2, jnp.zeros((B * m1p, 13), jnp.float32)], axis=1)
    g2 = _sc_gather(tab2, idx2)
    pos3 = g2[:, :3]
    hp3 = jnp.concatenate([g2[:, 3:35], pos3], axis=1)

    pcol3, prow3, _ = _pack_level(pos3, m2p)
    h3 = _level(hp3, pcol3, prow3, m2p, m2, k3, W3a, b3a, W3b, b3b)

    return _final(h3, Wr, br.reshape(1, 10), m2p, m2)
